# baseline (speedup 1.0000x reference)
import functools

import jax
import jax.numpy as jnp
from jax import lax
from jax.experimental import pallas as pl
from jax.experimental.pallas import tpu as pltpu
from jax.experimental.pallas import tpu_sc as plsc

N_NODES = 10000
N_EDGES = 160000
ROWS_PER_SUB = 640
N_PAD = ROWS_PER_SUB * 16
E_PAD = 163840
CHUNK = 128
BM = 1000
TAIL = N_NODES - 15 * ROWS_PER_SUB


def _sc_mesh():
    return plsc.VectorSubcoreMesh(core_axis_name="c", subcore_axis_name="s")


NBUF = 2


def _sc_scratch():
    return (
        [pltpu.VMEM((CHUNK,), jnp.int32) for _ in range(NBUF)]
        + [pltpu.VMEM((CHUNK,), jnp.int32) for _ in range(NBUF)]
        + [pltpu.VMEM((CHUNK, 128), jnp.float32)
           for _ in range(NBUF)]
        + [pltpu.VMEM_SHARED((N_PAD, 128), jnp.float32)]
        + [pltpu.SemaphoreType.DMA for _ in range(NBUF)]
        + [pltpu.SemaphoreType.DMA for _ in range(NBUF)]
    )


def _edge_pipeline(g_hbm, src_hbm, dst_hbm, acc, sidx, didx, rows,
                   gsem, ssem, ebase0, n_chunks):
    eb = pl.multiple_of(ebase0, CHUNK)
    pltpu.sync_copy(src_hbm.at[pl.ds(eb, CHUNK)], sidx[0])
    pltpu.sync_copy(dst_hbm.at[pl.ds(eb, CHUNK)], didx[0])
    pltpu.async_copy(g_hbm.at[sidx[0]], rows[0], gsem[0])

    def body2(q, carry):
        for b in range(2):
            j = q * 2 + b
            nb = 1 - b

            @pl.when(j + 1 < n_chunks)
            def _():
                eb = pl.multiple_of(ebase0 + (j + 1) * CHUNK, CHUNK)
                pltpu.sync_copy(src_hbm.at[pl.ds(eb, CHUNK)], sidx[nb])
                pltpu.sync_copy(dst_hbm.at[pl.ds(eb, CHUNK)], didx[nb])
                pltpu.async_copy(g_hbm.at[sidx[nb]], rows[nb], gsem[nb])

            pltpu.make_async_copy(g_hbm.at[sidx[b]], rows[b], gsem[b]).wait()
            pltpu.sync_copy(rows[b], acc.at[didx[b]], add=True)
        return carry

    lax.fori_loop(0, n_chunks // 2, body2, 0)


def _init_acc(table_hbm, acc, t):
    base = t * ROWS_PER_SUB

    @pl.when(t != 15)
    def _():
        pltpu.sync_copy(table_hbm.at[pl.ds(base, ROWS_PER_SUB)],
                        acc.at[pl.ds(base, ROWS_PER_SUB)])

    @pl.when(t == 15)
    def _():
        pltpu.sync_copy(table_hbm.at[pl.ds(15 * ROWS_PER_SUB, TAIL)],
                        acc.at[pl.ds(15 * ROWS_PER_SUB, TAIL)])


def _drain_acc(acc, out_slice, t):
    base = t * ROWS_PER_SUB

    @pl.when(t != 15)
    def _():
        pltpu.sync_copy(acc.at[pl.ds(base, ROWS_PER_SUB)],
                        out_slice.at[pl.ds(base, ROWS_PER_SUB)])

    @pl.when(t == 15)
    def _():
        pltpu.sync_copy(acc.at[pl.ds(15 * ROWS_PER_SUB, TAIL)],
                        out_slice.at[pl.ds(15 * ROWS_PER_SUB, TAIL)])


def _make_scatter_full():
    out_type = jax.ShapeDtypeStruct((2, N_NODES, 128), jnp.float32)
    epc = E_PAD // 16
    n_chunks = epc // CHUNK

    @functools.partial(pl.kernel, mesh=_sc_mesh(), out_type=out_type,
                       scratch_types=_sc_scratch())
    def sc_kernel(g0_hbm, g1_hbm, src_hbm, dst_hbm, out_hbm, *sc_refs):
        sidx = sc_refs[0:NBUF]
        didx = sc_refs[NBUF:2 * NBUF]
        rows = sc_refs[2 * NBUF:3 * NBUF]
        acc = sc_refs[3 * NBUF]
        gsem = sc_refs[3 * NBUF + 1:4 * NBUF + 1]
        ssem = sc_refs[4 * NBUF + 1:5 * NBUF + 1]

        c = lax.axis_index("c")
        t = lax.axis_index("s")
        ebase0 = t * epc

        for h, g_hbm in enumerate((g0_hbm, g1_hbm)):
            @pl.when(c == h)
            def _():
                _init_acc(g_hbm, acc, t)
                plsc.subcore_barrier()
                _edge_pipeline(g_hbm, src_hbm, dst_hbm, acc, sidx, didx,
                               rows, gsem, ssem, ebase0, n_chunks)
                plsc.subcore_barrier()
                _drain_acc(acc, out_hbm.at[h], t)

    return sc_kernel


def _make_scatter_partial():
    out_type = jax.ShapeDtypeStruct((2, N_NODES, 128), jnp.float32)
    epw = E_PAD // 32
    n_chunks = epw // CHUNK

    @functools.partial(pl.kernel, mesh=_sc_mesh(), out_type=out_type,
                       scratch_types=_sc_scratch())
    def sc_kernel(g_hbm, src_hbm, dst_hbm, zeros_hbm, out_hbm, *sc_refs):
        sidx = sc_refs[0:NBUF]
        didx = sc_refs[NBUF:2 * NBUF]
        rows = sc_refs[2 * NBUF:3 * NBUF]
        acc = sc_refs[3 * NBUF]
        gsem = sc_refs[3 * NBUF + 1:4 * NBUF + 1]
        ssem = sc_refs[4 * NBUF + 1:5 * NBUF + 1]

        c = lax.axis_index("c")
        t = lax.axis_index("s")
        ebase0 = (c * 16 + t) * epw

        @pl.when(c == 0)
        def _():
            _init_acc(g_hbm, acc, t)

        @pl.when(c == 1)
        def _():
            _init_acc(zeros_hbm, acc, t)

        plsc.subcore_barrier()
        _edge_pipeline(g_hbm, src_hbm, dst_hbm, acc, sidx, didx, rows,
                       gsem, ssem, ebase0, n_chunks)
        plsc.subcore_barrier()
        _drain_acc(acc, out_hbm.at[c], t)

    return sc_kernel


def _bspec(shape, blocked_m=True):
    if blocked_m:
        return pl.BlockSpec(shape, lambda i: (i,) + (0,) * (len(shape) - 1))
    return pl.BlockSpec(shape, lambda i: (0,) * len(shape))


def _make_tc_first():
    def body(x_ref, w_ref, da_ref, db_ref, o0_ref, o1_ref):
        s = lax.rsqrt(da_ref[...] + db_ref[...])
        g = jnp.dot(x_ref[...], w_ref[...],
                    preferred_element_type=jnp.float32) * s
        o0_ref[...] = g[:, :128]
        o1_ref[...] = g[:, 128:]

    return pl.pallas_call(
        body,
        grid=(N_NODES // BM,),
        in_specs=[
            _bspec((BM, 256)),
            _bspec((256, 256), blocked_m=False),
            _bspec((BM, 1)),
            _bspec((BM, 1)),
        ],
        out_specs=[_bspec((BM, 128)), _bspec((BM, 128))],
        out_shape=[jax.ShapeDtypeStruct((N_NODES, 128), jnp.float32)] * 2,
    )


def _make_tc_step256(dout, relu):
    h_out = dout // 128

    def body(p_ref, da_ref, db_ref, b_ref, w_ref, *o_refs):
        s = lax.rsqrt(da_ref[...] + db_ref[...])
        p = p_ref[...]
        t = s * jnp.concatenate([p[0], p[1]], axis=1) + b_ref[...]
        if relu:
            t = jnp.maximum(t, 0.0)
        g = jnp.dot(t, w_ref[...], preferred_element_type=jnp.float32) * s
        for h in range(h_out):
            o_refs[h][...] = g[:, h * 128:(h + 1) * 128]

    return pl.pallas_call(
        body,
        grid=(N_NODES // BM,),
        in_specs=[
            pl.BlockSpec((2, BM, 128), lambda i: (0, i, 0)),
            _bspec((BM, 1)),
            _bspec((BM, 1)),
            _bspec((1, 256), blocked_m=False),
            _bspec((256, dout), blocked_m=False),
        ],
        out_specs=[_bspec((BM, 128))] * h_out,
        out_shape=[jax.ShapeDtypeStruct((N_NODES, 128), jnp.float32)] * h_out,
    )


def _make_tc_step128(relu):
    def body(p_ref, da_ref, db_ref, b_ref, w_ref, o0_ref, o1_ref):
        s = lax.rsqrt(da_ref[...] + db_ref[...])
        p = p_ref[...]
        t = s * (p[0] + p[1]) + b_ref[...]
        if relu:
            t = jnp.maximum(t, 0.0)
        g = jnp.dot(t, w_ref[...], preferred_element_type=jnp.float32) * s
        o0_ref[...] = g[:, :128]
        o1_ref[...] = g[:, 128:]

    return pl.pallas_call(
        body,
        grid=(N_NODES // BM,),
        in_specs=[
            pl.BlockSpec((2, BM, 128), lambda i: (0, i, 0)),
            _bspec((BM, 1)),
            _bspec((BM, 1)),
            _bspec((1, 128), blocked_m=False),
            _bspec((128, 256), blocked_m=False),
        ],
        out_specs=[_bspec((BM, 128)), _bspec((BM, 128))],
        out_shape=[jax.ShapeDtypeStruct((N_NODES, 128), jnp.float32)] * 2,
    )


def _make_tc_final():
    def body(p_ref, da_ref, db_ref, b_ref, o_ref):
        s = lax.rsqrt(da_ref[...] + db_ref[...])
        p = p_ref[...]
        o_ref[...] = s * jnp.concatenate([p[0], p[1]], axis=1) + b_ref[...]

    return pl.pallas_call(
        body,
        grid=(N_NODES // BM,),
        in_specs=[
            pl.BlockSpec((2, BM, 128), lambda i: (0, i, 0)),
            _bspec((BM, 1)),
            _bspec((BM, 1)),
            _bspec((1, 256), blocked_m=False),
        ],
        out_specs=_bspec((BM, 256)),
        out_shape=jax.ShapeDtypeStruct((N_NODES, 256), jnp.float32),
    )


def kernel(x, edge_index, W1, b1, W2, b2, W3, b3, W4, b4, W5, b5, W6, b6):
    src = edge_index[0].astype(jnp.int32)
    dst = edge_index[1].astype(jnp.int32)
    e_pad = E_PAD - src.shape[0]
    src_p = jnp.concatenate([src, jnp.zeros((e_pad,), jnp.int32)])
    dst_p = jnp.concatenate([dst, jnp.full((e_pad,), N_NODES, jnp.int32)])

    ones128 = jnp.ones((N_NODES, 128), jnp.float32)
    zeros128 = jnp.zeros((N_NODES, 128), jnp.float32)

    sc_full = _make_scatter_full()
    sc_part = _make_scatter_partial()
    tc_first = _make_tc_first()
    tc_256_256 = _make_tc_step256(256, relu=True)
    tc_256_128 = _make_tc_step256(128, relu=True)
    tc_128_256 = _make_tc_step128(relu=False)
    tc_final = _make_tc_final()

    deg = sc_part(ones128, src_p, dst_p, zeros128)
    da = deg[0, :, 0:1]
    db = deg[1, :, 0:1]

    g0, g1 = tc_first(x, W1, da, db)
    p = sc_full(g0, g1, src_p, dst_p)
    g0, g1 = tc_256_256(p, da, db, b1.reshape(1, -1), W2)
    p = sc_full(g0, g1, src_p, dst_p)
    (g3,) = tc_256_128(p, da, db, b2.reshape(1, -1), W3)
    q = sc_part(g3, src_p, dst_p, zeros128)
    g0, g1 = tc_128_256(q, da, db, b3.reshape(1, -1), W4)
    p = sc_full(g0, g1, src_p, dst_p)
    g0, g1 = tc_256_256(p, da, db, b4.reshape(1, -1), W5)
    p = sc_full(g0, g1, src_p, dst_p)
    g0, g1 = tc_256_256(p, da, db, b5.reshape(1, -1), W6)
    p = sc_full(g0, g1, src_p, dst_p)
    out = tc_final(p, da, db, b6.reshape(1, -1))
    return out


# speedup vs baseline: 1.0216x; 1.0216x over previous
import functools

import jax
import jax.numpy as jnp
from jax import lax
from jax.experimental import pallas as pl
from jax.experimental.pallas import tpu as pltpu
from jax.experimental.pallas import tpu_sc as plsc

N_NODES = 10000
N_EDGES = 160000
ROWS_PER_SUB = 640
N_PAD = ROWS_PER_SUB * 16
E_PAD = 163840
CHUNK = 80
BM = 1000
TAIL = N_NODES - 15 * ROWS_PER_SUB


def _sc_mesh():
    return plsc.VectorSubcoreMesh(core_axis_name="c", subcore_axis_name="s")


NBUF = 4


def _sc_scratch():
    return (
        [pltpu.VMEM((CHUNK,), jnp.int32) for _ in range(NBUF)]
        + [pltpu.VMEM((CHUNK,), jnp.int32) for _ in range(NBUF)]
        + [pltpu.VMEM((CHUNK, 128), jnp.float32)
           for _ in range(NBUF)]
        + [pltpu.VMEM_SHARED((N_PAD, 128), jnp.float32)]
        + [pltpu.SemaphoreType.DMA for _ in range(NBUF)]
        + [pltpu.SemaphoreType.DMA for _ in range(NBUF)]
    )


def _edge_pipeline(g_hbm, src_hbm, dst_hbm, acc, sidx, didx, rows,
                   gsem, ssem, ebase0, n_chunks):
    for b in range(2):
        eb = pl.multiple_of(ebase0 + b * CHUNK, CHUNK)
        pltpu.sync_copy(src_hbm.at[pl.ds(eb, CHUNK)], sidx[b])
        pltpu.sync_copy(dst_hbm.at[pl.ds(eb, CHUNK)], didx[b])
        pltpu.async_copy(g_hbm.at[sidx[b]], rows[b], gsem[b])

    def body4(q, carry):
        for r in range(4):
            j = q * 4 + r
            pb = (r + 2) % 4

            @pl.when(jnp.logical_and(j >= 2, j + 2 < n_chunks))
            def _():
                pltpu.make_async_copy(rows[pb], acc.at[didx[pb]],
                                      ssem[pb]).wait()

            @pl.when(j + 2 < n_chunks)
            def _():
                eb = pl.multiple_of(ebase0 + (j + 2) * CHUNK, CHUNK)
                pltpu.sync_copy(src_hbm.at[pl.ds(eb, CHUNK)], sidx[pb])
                pltpu.sync_copy(dst_hbm.at[pl.ds(eb, CHUNK)], didx[pb])
                pltpu.async_copy(g_hbm.at[sidx[pb]], rows[pb], gsem[pb])

            pltpu.make_async_copy(g_hbm.at[sidx[r]], rows[r], gsem[r]).wait()
            pltpu.async_copy(rows[r], acc.at[didx[r]], ssem[r], add=True)
        return carry

    lax.fori_loop(0, n_chunks // 4, body4, 0)
    for b in range(4):
        pltpu.make_async_copy(rows[b], acc.at[didx[b]], ssem[b]).wait()


def _init_acc(table_hbm, acc, t):
    base = t * ROWS_PER_SUB

    @pl.when(t != 15)
    def _():
        pltpu.sync_copy(table_hbm.at[pl.ds(base, ROWS_PER_SUB)],
                        acc.at[pl.ds(base, ROWS_PER_SUB)])

    @pl.when(t == 15)
    def _():
        pltpu.sync_copy(table_hbm.at[pl.ds(15 * ROWS_PER_SUB, TAIL)],
                        acc.at[pl.ds(15 * ROWS_PER_SUB, TAIL)])


def _drain_acc(acc, out_slice, t):
    base = t * ROWS_PER_SUB

    @pl.when(t != 15)
    def _():
        pltpu.sync_copy(acc.at[pl.ds(base, ROWS_PER_SUB)],
                        out_slice.at[pl.ds(base, ROWS_PER_SUB)])

    @pl.when(t == 15)
    def _():
        pltpu.sync_copy(acc.at[pl.ds(15 * ROWS_PER_SUB, TAIL)],
                        out_slice.at[pl.ds(15 * ROWS_PER_SUB, TAIL)])


def _make_scatter_full():
    out_type = jax.ShapeDtypeStruct((2, N_NODES, 128), jnp.float32)
    epc = E_PAD // 16
    n_chunks = epc // CHUNK

    @functools.partial(pl.kernel, mesh=_sc_mesh(), out_type=out_type,
                       scratch_types=_sc_scratch())
    def sc_kernel(g0_hbm, g1_hbm, src_hbm, dst_hbm, out_hbm, *sc_refs):
        sidx = sc_refs[0:NBUF]
        didx = sc_refs[NBUF:2 * NBUF]
        rows = sc_refs[2 * NBUF:3 * NBUF]
        acc = sc_refs[3 * NBUF]
        gsem = sc_refs[3 * NBUF + 1:4 * NBUF + 1]
        ssem = sc_refs[4 * NBUF + 1:5 * NBUF + 1]

        c = lax.axis_index("c")
        t = lax.axis_index("s")
        ebase0 = t * epc

        for h, g_hbm in enumerate((g0_hbm, g1_hbm)):
            @pl.when(c == h)
            def _():
                _init_acc(g_hbm, acc, t)
                plsc.subcore_barrier()
                _edge_pipeline(g_hbm, src_hbm, dst_hbm, acc, sidx, didx,
                               rows, gsem, ssem, ebase0, n_chunks)
                plsc.subcore_barrier()
                _drain_acc(acc, out_hbm.at[h], t)

    return sc_kernel


def _make_scatter_partial():
    out_type = jax.ShapeDtypeStruct((2, N_NODES, 128), jnp.float32)
    epw = E_PAD // 32
    n_chunks = epw // CHUNK

    @functools.partial(pl.kernel, mesh=_sc_mesh(), out_type=out_type,
                       scratch_types=_sc_scratch())
    def sc_kernel(g_hbm, src_hbm, dst_hbm, zeros_hbm, out_hbm, *sc_refs):
        sidx = sc_refs[0:NBUF]
        didx = sc_refs[NBUF:2 * NBUF]
        rows = sc_refs[2 * NBUF:3 * NBUF]
        acc = sc_refs[3 * NBUF]
        gsem = sc_refs[3 * NBUF + 1:4 * NBUF + 1]
        ssem = sc_refs[4 * NBUF + 1:5 * NBUF + 1]

        c = lax.axis_index("c")
        t = lax.axis_index("s")
        ebase0 = (c * 16 + t) * epw

        @pl.when(c == 0)
        def _():
            _init_acc(g_hbm, acc, t)

        @pl.when(c == 1)
        def _():
            _init_acc(zeros_hbm, acc, t)

        plsc.subcore_barrier()
        _edge_pipeline(g_hbm, src_hbm, dst_hbm, acc, sidx, didx, rows,
                       gsem, ssem, ebase0, n_chunks)
        plsc.subcore_barrier()
        _drain_acc(acc, out_hbm.at[c], t)

    return sc_kernel


def _bspec(shape, blocked_m=True):
    if blocked_m:
        return pl.BlockSpec(shape, lambda i: (i,) + (0,) * (len(shape) - 1))
    return pl.BlockSpec(shape, lambda i: (0,) * len(shape))


def _make_tc_first():
    def body(x_ref, w_ref, da_ref, db_ref, o0_ref, o1_ref):
        s = lax.rsqrt(da_ref[...] + db_ref[...])
        g = jnp.dot(x_ref[...], w_ref[...],
                    preferred_element_type=jnp.float32) * s
        o0_ref[...] = g[:, :128]
        o1_ref[...] = g[:, 128:]

    return pl.pallas_call(
        body,
        grid=(N_NODES // BM,),
        in_specs=[
            _bspec((BM, 256)),
            _bspec((256, 256), blocked_m=False),
            _bspec((BM, 1)),
            _bspec((BM, 1)),
        ],
        out_specs=[_bspec((BM, 128)), _bspec((BM, 128))],
        out_shape=[jax.ShapeDtypeStruct((N_NODES, 128), jnp.float32)] * 2,
    )


def _make_tc_step256(dout, relu):
    h_out = dout // 128

    def body(p_ref, da_ref, db_ref, b_ref, w_ref, *o_refs):
        s = lax.rsqrt(da_ref[...] + db_ref[...])
        p = p_ref[...]
        t = s * jnp.concatenate([p[0], p[1]], axis=1) + b_ref[...]
        if relu:
            t = jnp.maximum(t, 0.0)
        g = jnp.dot(t, w_ref[...], preferred_element_type=jnp.float32) * s
        for h in range(h_out):
            o_refs[h][...] = g[:, h * 128:(h + 1) * 128]

    return pl.pallas_call(
        body,
        grid=(N_NODES // BM,),
        in_specs=[
            pl.BlockSpec((2, BM, 128), lambda i: (0, i, 0)),
            _bspec((BM, 1)),
            _bspec((BM, 1)),
            _bspec((1, 256), blocked_m=False),
            _bspec((256, dout), blocked_m=False),
        ],
        out_specs=[_bspec((BM, 128))] * h_out,
        out_shape=[jax.ShapeDtypeStruct((N_NODES, 128), jnp.float32)] * h_out,
    )


def _make_tc_step128(relu):
    def body(p_ref, da_ref, db_ref, b_ref, w_ref, o0_ref, o1_ref):
        s = lax.rsqrt(da_ref[...] + db_ref[...])
        p = p_ref[...]
        t = s * (p[0] + p[1]) + b_ref[...]
        if relu:
            t = jnp.maximum(t, 0.0)
        g = jnp.dot(t, w_ref[...], preferred_element_type=jnp.float32) * s
        o0_ref[...] = g[:, :128]
        o1_ref[...] = g[:, 128:]

    return pl.pallas_call(
        body,
        grid=(N_NODES // BM,),
        in_specs=[
            pl.BlockSpec((2, BM, 128), lambda i: (0, i, 0)),
            _bspec((BM, 1)),
            _bspec((BM, 1)),
            _bspec((1, 128), blocked_m=False),
            _bspec((128, 256), blocked_m=False),
        ],
        out_specs=[_bspec((BM, 128)), _bspec((BM, 128))],
        out_shape=[jax.ShapeDtypeStruct((N_NODES, 128), jnp.float32)] * 2,
    )


def _make_tc_final():
    def body(p_ref, da_ref, db_ref, b_ref, o_ref):
        s = lax.rsqrt(da_ref[...] + db_ref[...])
        p = p_ref[...]
        o_ref[...] = s * jnp.concatenate([p[0], p[1]], axis=1) + b_ref[...]

    return pl.pallas_call(
        body,
        grid=(N_NODES // BM,),
        in_specs=[
            pl.BlockSpec((2, BM, 128), lambda i: (0, i, 0)),
            _bspec((BM, 1)),
            _bspec((BM, 1)),
            _bspec((1, 256), blocked_m=False),
        ],
        out_specs=_bspec((BM, 256)),
        out_shape=jax.ShapeDtypeStruct((N_NODES, 256), jnp.float32),
    )


def kernel(x, edge_index, W1, b1, W2, b2, W3, b3, W4, b4, W5, b5, W6, b6):
    src = edge_index[0].astype(jnp.int32)
    dst = edge_index[1].astype(jnp.int32)
    e_pad = E_PAD - src.shape[0]
    src_p = jnp.concatenate([src, jnp.zeros((e_pad,), jnp.int32)])
    dst_p = jnp.concatenate([dst, jnp.full((e_pad,), N_NODES, jnp.int32)])

    ones128 = jnp.ones((N_NODES, 128), jnp.float32)
    zeros128 = jnp.zeros((N_NODES, 128), jnp.float32)

    sc_full = _make_scatter_full()
    sc_part = _make_scatter_partial()
    tc_first = _make_tc_first()
    tc_256_256 = _make_tc_step256(256, relu=True)
    tc_256_128 = _make_tc_step256(128, relu=True)
    tc_128_256 = _make_tc_step128(relu=False)
    tc_final = _make_tc_final()

    deg = sc_part(ones128, src_p, dst_p, zeros128)
    da = deg[0, :, 0:1]
    db = deg[1, :, 0:1]

    g0, g1 = tc_first(x, W1, da, db)
    p = sc_full(g0, g1, src_p, dst_p)
    g0, g1 = tc_256_256(p, da, db, b1.reshape(1, -1), W2)
    p = sc_full(g0, g1, src_p, dst_p)
    (g3,) = tc_256_128(p, da, db, b2.reshape(1, -1), W3)
    q = sc_part(g3, src_p, dst_p, zeros128)
    g0, g1 = tc_128_256(q, da, db, b3.reshape(1, -1), W4)
    p = sc_full(g0, g1, src_p, dst_p)
    g0, g1 = tc_256_256(p, da, db, b4.reshape(1, -1), W5)
    p = sc_full(g0, g1, src_p, dst_p)
    g0, g1 = tc_256_256(p, da, db, b5.reshape(1, -1), W6)
    p = sc_full(g0, g1, src_p, dst_p)
    out = tc_final(p, da, db, b6.reshape(1, -1))
    return out


# speedup vs baseline: 2.3908x; 2.3401x over previous
import functools

import jax
import jax.numpy as jnp
from jax import lax
from jax.experimental import pallas as pl
from jax.experimental.pallas import tpu as pltpu
from jax.experimental.pallas import tpu_sc as plsc

N_NODES = 10000
N_EDGES = 160000
ROWS_PER_SUB = 640
N_PAD = ROWS_PER_SUB * 16
E_PAD = 163840
CHUNK = 80
BM = 1000
TAIL = N_NODES - 15 * ROWS_PER_SUB


def _sc_mesh():
    return plsc.VectorSubcoreMesh(core_axis_name="c", subcore_axis_name="s")


NBUF = 4


def _sc_scratch():
    return (
        [pltpu.VMEM((CHUNK,), jnp.int32) for _ in range(NBUF)]
        + [pltpu.VMEM((CHUNK,), jnp.int32) for _ in range(NBUF)]
        + [pltpu.VMEM((CHUNK, 128), jnp.float32)
           for _ in range(NBUF)]
        + [pltpu.VMEM_SHARED((N_PAD, 128), jnp.float32)]
        + [pltpu.SemaphoreType.DMA for _ in range(NBUF)]
        + [pltpu.SemaphoreType.DMA for _ in range(NBUF)]
    )


def _edge_pipeline(g_hbm, src_hbm, dst_hbm, acc, sidx, didx, rows,
                   gsem, ssem, ebase0, n_chunks):
    for b in range(2):
        eb = pl.multiple_of(ebase0 + b * CHUNK, CHUNK)
        pltpu.sync_copy(src_hbm.at[pl.ds(eb, CHUNK)], sidx[b])
        pltpu.sync_copy(dst_hbm.at[pl.ds(eb, CHUNK)], didx[b])
        pltpu.async_copy(g_hbm.at[sidx[b]], rows[b], gsem[b])

    def body4(q, carry):
        for r in range(4):
            j = q * 4 + r
            pb = (r + 2) % 4

            @pl.when(jnp.logical_and(j >= 2, j + 2 < n_chunks))
            def _():
                pltpu.make_async_copy(rows[pb], acc.at[didx[pb]],
                                      ssem[pb]).wait()

            @pl.when(j + 2 < n_chunks)
            def _():
                eb = pl.multiple_of(ebase0 + (j + 2) * CHUNK, CHUNK)
                pltpu.sync_copy(src_hbm.at[pl.ds(eb, CHUNK)], sidx[pb])
                pltpu.sync_copy(dst_hbm.at[pl.ds(eb, CHUNK)], didx[pb])
                pltpu.async_copy(g_hbm.at[sidx[pb]], rows[pb], gsem[pb])

            pltpu.make_async_copy(g_hbm.at[sidx[r]], rows[r], gsem[r]).wait()
            pltpu.async_copy(rows[r], acc.at[didx[r]], ssem[r], add=True)
        return carry

    lax.fori_loop(0, n_chunks // 4, body4, 0)
    for b in range(4):
        pltpu.make_async_copy(rows[b], acc.at[didx[b]], ssem[b]).wait()


def _init_acc(table_hbm, acc, t):
    base = t * ROWS_PER_SUB

    @pl.when(t != 15)
    def _():
        pltpu.sync_copy(table_hbm.at[pl.ds(base, ROWS_PER_SUB)],
                        acc.at[pl.ds(base, ROWS_PER_SUB)])

    @pl.when(t == 15)
    def _():
        pltpu.sync_copy(table_hbm.at[pl.ds(15 * ROWS_PER_SUB, TAIL)],
                        acc.at[pl.ds(15 * ROWS_PER_SUB, TAIL)])


def _drain_acc(acc, out_slice, t):
    base = t * ROWS_PER_SUB

    @pl.when(t != 15)
    def _():
        pltpu.sync_copy(acc.at[pl.ds(base, ROWS_PER_SUB)],
                        out_slice.at[pl.ds(base, ROWS_PER_SUB)])

    @pl.when(t == 15)
    def _():
        pltpu.sync_copy(acc.at[pl.ds(15 * ROWS_PER_SUB, TAIL)],
                        out_slice.at[pl.ds(15 * ROWS_PER_SUB, TAIL)])


def _make_scatter_full():
    out_type = jax.ShapeDtypeStruct((2, N_NODES, 128), jnp.float32)
    epc = E_PAD // 16
    n_chunks = epc // CHUNK

    @functools.partial(pl.kernel, mesh=_sc_mesh(), out_type=out_type,
                       scratch_types=_sc_scratch())
    def sc_kernel(g0_hbm, g1_hbm, src_hbm, dst_hbm, out_hbm, *sc_refs):
        sidx = sc_refs[0:NBUF]
        didx = sc_refs[NBUF:2 * NBUF]
        rows = sc_refs[2 * NBUF:3 * NBUF]
        acc = sc_refs[3 * NBUF]
        gsem = sc_refs[3 * NBUF + 1:4 * NBUF + 1]
        ssem = sc_refs[4 * NBUF + 1:5 * NBUF + 1]

        c = lax.axis_index("c")
        t = lax.axis_index("s")
        ebase0 = t * epc

        for h, g_hbm in enumerate((g0_hbm, g1_hbm)):
            @pl.when(c == h)
            def _():
                _init_acc(g_hbm, acc, t)
                plsc.subcore_barrier()
                _edge_pipeline(g_hbm, src_hbm, dst_hbm, acc, sidx, didx,
                               rows, gsem, ssem, ebase0, n_chunks)
                plsc.subcore_barrier()
                _drain_acc(acc, out_hbm.at[h], t)

    return sc_kernel


def _make_scatter_partial():
    out_type = jax.ShapeDtypeStruct((2, N_NODES, 128), jnp.float32)
    epw = E_PAD // 32
    n_chunks = epw // CHUNK

    @functools.partial(pl.kernel, mesh=_sc_mesh(), out_type=out_type,
                       scratch_types=_sc_scratch())
    def sc_kernel(g_hbm, src_hbm, dst_hbm, zeros_hbm, out_hbm, *sc_refs):
        sidx = sc_refs[0:NBUF]
        didx = sc_refs[NBUF:2 * NBUF]
        rows = sc_refs[2 * NBUF:3 * NBUF]
        acc = sc_refs[3 * NBUF]
        gsem = sc_refs[3 * NBUF + 1:4 * NBUF + 1]
        ssem = sc_refs[4 * NBUF + 1:5 * NBUF + 1]

        c = lax.axis_index("c")
        t = lax.axis_index("s")
        ebase0 = (c * 16 + t) * epw

        @pl.when(c == 0)
        def _():
            _init_acc(g_hbm, acc, t)

        @pl.when(c == 1)
        def _():
            _init_acc(zeros_hbm, acc, t)

        plsc.subcore_barrier()
        _edge_pipeline(g_hbm, src_hbm, dst_hbm, acc, sidx, didx, rows,
                       gsem, ssem, ebase0, n_chunks)
        plsc.subcore_barrier()
        _drain_acc(acc, out_hbm.at[c], t)

    return sc_kernel


def _bspec(shape, blocked_m=True):
    if blocked_m:
        return pl.BlockSpec(shape, lambda i: (i,) + (0,) * (len(shape) - 1))
    return pl.BlockSpec(shape, lambda i: (0,) * len(shape))


def _make_tc_first():
    def body(x_ref, w_ref, da_ref, db_ref, o0_ref, o1_ref):
        s = lax.rsqrt(da_ref[...] + db_ref[...])
        g = jnp.dot(x_ref[...], w_ref[...],
                    preferred_element_type=jnp.float32) * s
        o0_ref[...] = g[:, :128]
        o1_ref[...] = g[:, 128:]

    return pl.pallas_call(
        body,
        grid=(N_NODES // BM,),
        in_specs=[
            _bspec((BM, 256)),
            _bspec((256, 256), blocked_m=False),
            _bspec((BM, 1)),
            _bspec((BM, 1)),
        ],
        out_specs=[_bspec((BM, 128)), _bspec((BM, 128))],
        out_shape=[jax.ShapeDtypeStruct((N_NODES, 128), jnp.float32)] * 2,
    )


def _make_tc_step256(dout, relu):
    h_out = dout // 128

    def body(p_ref, da_ref, db_ref, b_ref, w_ref, *o_refs):
        s = lax.rsqrt(da_ref[...] + db_ref[...])
        p = p_ref[...]
        t = s * jnp.concatenate([p[0], p[1]], axis=1) + b_ref[...]
        if relu:
            t = jnp.maximum(t, 0.0)
        g = jnp.dot(t, w_ref[...], preferred_element_type=jnp.float32) * s
        for h in range(h_out):
            o_refs[h][...] = g[:, h * 128:(h + 1) * 128]

    return pl.pallas_call(
        body,
        grid=(N_NODES // BM,),
        in_specs=[
            pl.BlockSpec((2, BM, 128), lambda i: (0, i, 0)),
            _bspec((BM, 1)),
            _bspec((BM, 1)),
            _bspec((1, 256), blocked_m=False),
            _bspec((256, dout), blocked_m=False),
        ],
        out_specs=[_bspec((BM, 128))] * h_out,
        out_shape=[jax.ShapeDtypeStruct((N_NODES, 128), jnp.float32)] * h_out,
    )


def _make_tc_step128(relu):
    def body(p_ref, da_ref, db_ref, b_ref, w_ref, o0_ref, o1_ref):
        s = lax.rsqrt(da_ref[...] + db_ref[...])
        p = p_ref[...]
        t = s * (p[0] + p[1]) + b_ref[...]
        if relu:
            t = jnp.maximum(t, 0.0)
        g = jnp.dot(t, w_ref[...], preferred_element_type=jnp.float32) * s
        o0_ref[...] = g[:, :128]
        o1_ref[...] = g[:, 128:]

    return pl.pallas_call(
        body,
        grid=(N_NODES // BM,),
        in_specs=[
            pl.BlockSpec((2, BM, 128), lambda i: (0, i, 0)),
            _bspec((BM, 1)),
            _bspec((BM, 1)),
            _bspec((1, 128), blocked_m=False),
            _bspec((128, 256), blocked_m=False),
        ],
        out_specs=[_bspec((BM, 128)), _bspec((BM, 128))],
        out_shape=[jax.ShapeDtypeStruct((N_NODES, 128), jnp.float32)] * 2,
    )


def _make_tc_final():
    def body(p_ref, da_ref, db_ref, b_ref, o_ref):
        s = lax.rsqrt(da_ref[...] + db_ref[...])
        p = p_ref[...]
        o_ref[...] = s * jnp.concatenate([p[0], p[1]], axis=1) + b_ref[...]

    return pl.pallas_call(
        body,
        grid=(N_NODES // BM,),
        in_specs=[
            pl.BlockSpec((2, BM, 128), lambda i: (0, i, 0)),
            _bspec((BM, 1)),
            _bspec((BM, 1)),
            _bspec((1, 256), blocked_m=False),
        ],
        out_specs=_bspec((BM, 256)),
        out_shape=jax.ShapeDtypeStruct((N_NODES, 256), jnp.float32),
    )


def kernel(x, edge_index, W1, b1, W2, b2, W3, b3, W4, b4, W5, b5, W6, b6):
    src = edge_index[0].astype(jnp.int32)
    dst = edge_index[1].astype(jnp.int32)
    e_pad = E_PAD - src.shape[0]
    src_p = jnp.concatenate(
        [src, (jnp.arange(e_pad, dtype=jnp.int32) * 64) % N_NODES])
    dst_p = jnp.concatenate([dst, jnp.full((e_pad,), N_NODES, jnp.int32)])

    ones128 = jnp.ones((N_NODES, 128), jnp.float32)
    zeros128 = jnp.zeros((N_NODES, 128), jnp.float32)

    sc_full = _make_scatter_full()
    sc_part = _make_scatter_partial()
    tc_first = _make_tc_first()
    tc_256_256 = _make_tc_step256(256, relu=True)
    tc_256_128 = _make_tc_step256(128, relu=True)
    tc_128_256 = _make_tc_step128(relu=False)
    tc_final = _make_tc_final()

    deg = sc_part(ones128, src_p, dst_p, zeros128)
    da = deg[0, :, 0:1]
    db = deg[1, :, 0:1]

    g0, g1 = tc_first(x, W1, da, db)
    p = sc_full(g0, g1, src_p, dst_p)
    g0, g1 = tc_256_256(p, da, db, b1.reshape(1, -1), W2)
    p = sc_full(g0, g1, src_p, dst_p)
    (g3,) = tc_256_128(p, da, db, b2.reshape(1, -1), W3)
    q = sc_part(g3, src_p, dst_p, zeros128)
    g0, g1 = tc_128_256(q, da, db, b3.reshape(1, -1), W4)
    p = sc_full(g0, g1, src_p, dst_p)
    g0, g1 = tc_256_256(p, da, db, b4.reshape(1, -1), W5)
    p = sc_full(g0, g1, src_p, dst_p)
    g0, g1 = tc_256_256(p, da, db, b5.reshape(1, -1), W6)
    p = sc_full(g0, g1, src_p, dst_p)
    out = tc_final(p, da, db, b6.reshape(1, -1))
    return out


# speedup vs baseline: 2.3963x; 1.0023x over previous
import functools

import jax
import jax.numpy as jnp
from jax import lax
from jax.experimental import pallas as pl
from jax.experimental.pallas import tpu as pltpu
from jax.experimental.pallas import tpu_sc as plsc

N_NODES = 10000
N_EDGES = 160000
ROWS_PER_SUB = 640
N_PAD = ROWS_PER_SUB * 16
E_PAD = 163840
CHUNK = 80
BM = 1000
TAIL = N_NODES - 15 * ROWS_PER_SUB


def _sc_mesh():
    return plsc.VectorSubcoreMesh(core_axis_name="c", subcore_axis_name="s")


NBUF = 4


def _sc_scratch():
    return (
        [pltpu.VMEM((CHUNK,), jnp.int32) for _ in range(NBUF)]
        + [pltpu.VMEM((CHUNK,), jnp.int32) for _ in range(NBUF)]
        + [pltpu.VMEM((CHUNK, 128), jnp.float32)
           for _ in range(NBUF)]
        + [pltpu.VMEM_SHARED((N_PAD, 128), jnp.float32)]
        + [pltpu.SemaphoreType.DMA for _ in range(NBUF)]
        + [pltpu.SemaphoreType.DMA for _ in range(NBUF)]
    )


def _edge_pipeline(g_hbm, src_hbm, dst_hbm, acc, sidx, didx, rows,
                   gsem, ssem, ebase0, n_chunks):
    for b in range(2):
        eb = pl.multiple_of(ebase0 + b * CHUNK, CHUNK)
        pltpu.sync_copy(src_hbm.at[pl.ds(eb, CHUNK)], sidx[b])
        pltpu.sync_copy(dst_hbm.at[pl.ds(eb, CHUNK)], didx[b])
        pltpu.async_copy(g_hbm.at[sidx[b]], rows[b], gsem[b])

    def body4(q, carry):
        for r in range(4):
            j = q * 4 + r
            pb = (r + 2) % 4

            @pl.when(jnp.logical_and(j >= 2, j + 2 < n_chunks))
            def _():
                pltpu.make_async_copy(rows[pb], acc.at[didx[pb]],
                                      ssem[pb]).wait()

            @pl.when(j + 2 < n_chunks)
            def _():
                eb = pl.multiple_of(ebase0 + (j + 2) * CHUNK, CHUNK)
                pltpu.sync_copy(src_hbm.at[pl.ds(eb, CHUNK)], sidx[pb])
                pltpu.sync_copy(dst_hbm.at[pl.ds(eb, CHUNK)], didx[pb])
                pltpu.async_copy(g_hbm.at[sidx[pb]], rows[pb], gsem[pb])

            pltpu.make_async_copy(g_hbm.at[sidx[r]], rows[r], gsem[r]).wait()
            pltpu.async_copy(rows[r], acc.at[didx[r]], ssem[r], add=True)
        return carry

    lax.fori_loop(0, n_chunks // 4, body4, 0)
    for b in range(4):
        pltpu.make_async_copy(rows[b], acc.at[didx[b]], ssem[b]).wait()


def _init_acc(table_hbm, acc, t):
    base = t * ROWS_PER_SUB

    @pl.when(t != 15)
    def _():
        pltpu.sync_copy(table_hbm.at[pl.ds(base, ROWS_PER_SUB)],
                        acc.at[pl.ds(base, ROWS_PER_SUB)])

    @pl.when(t == 15)
    def _():
        pltpu.sync_copy(table_hbm.at[pl.ds(15 * ROWS_PER_SUB, TAIL)],
                        acc.at[pl.ds(15 * ROWS_PER_SUB, TAIL)])


def _drain_acc(acc, out_slice, t):
    base = t * ROWS_PER_SUB

    @pl.when(t != 15)
    def _():
        pltpu.sync_copy(acc.at[pl.ds(base, ROWS_PER_SUB)],
                        out_slice.at[pl.ds(base, ROWS_PER_SUB)])

    @pl.when(t == 15)
    def _():
        pltpu.sync_copy(acc.at[pl.ds(15 * ROWS_PER_SUB, TAIL)],
                        out_slice.at[pl.ds(15 * ROWS_PER_SUB, TAIL)])


def _make_scatter_full():
    out_type = jax.ShapeDtypeStruct((2, N_NODES, 128), jnp.float32)
    epc = E_PAD // 16
    n_chunks = epc // CHUNK

    @functools.partial(pl.kernel, mesh=_sc_mesh(), out_type=out_type,
                       scratch_types=_sc_scratch())
    def sc_kernel(g0_hbm, g1_hbm, src_hbm, dst_hbm, out_hbm, *sc_refs):
        sidx = sc_refs[0:NBUF]
        didx = sc_refs[NBUF:2 * NBUF]
        rows = sc_refs[2 * NBUF:3 * NBUF]
        acc = sc_refs[3 * NBUF]
        gsem = sc_refs[3 * NBUF + 1:4 * NBUF + 1]
        ssem = sc_refs[4 * NBUF + 1:5 * NBUF + 1]

        c = lax.axis_index("c")
        t = lax.axis_index("s")
        ebase0 = t * epc

        for h, g_hbm in enumerate((g0_hbm, g1_hbm)):
            @pl.when(c == h)
            def _():
                _init_acc(g_hbm, acc, t)
                plsc.subcore_barrier()
                _edge_pipeline(g_hbm, src_hbm, dst_hbm, acc, sidx, didx,
                               rows, gsem, ssem, ebase0, n_chunks)
                plsc.subcore_barrier()
                _drain_acc(acc, out_hbm.at[h], t)

    return sc_kernel


def _make_scatter_partial():
    out_type = jax.ShapeDtypeStruct((2, N_NODES, 128), jnp.float32)
    epw = E_PAD // 32
    n_chunks = epw // CHUNK

    @functools.partial(pl.kernel, mesh=_sc_mesh(), out_type=out_type,
                       scratch_types=_sc_scratch())
    def sc_kernel(g_hbm, src_hbm, dst_hbm, zeros_hbm, out_hbm, *sc_refs):
        sidx = sc_refs[0:NBUF]
        didx = sc_refs[NBUF:2 * NBUF]
        rows = sc_refs[2 * NBUF:3 * NBUF]
        acc = sc_refs[3 * NBUF]
        gsem = sc_refs[3 * NBUF + 1:4 * NBUF + 1]
        ssem = sc_refs[4 * NBUF + 1:5 * NBUF + 1]

        c = lax.axis_index("c")
        t = lax.axis_index("s")
        ebase0 = (c * 16 + t) * epw

        @pl.when(c == 0)
        def _():
            _init_acc(g_hbm, acc, t)

        @pl.when(c == 1)
        def _():
            _init_acc(zeros_hbm, acc, t)

        plsc.subcore_barrier()
        _edge_pipeline(g_hbm, src_hbm, dst_hbm, acc, sidx, didx, rows,
                       gsem, ssem, ebase0, n_chunks)
        plsc.subcore_barrier()
        _drain_acc(acc, out_hbm.at[c], t)

    return sc_kernel


def _bspec(shape, blocked_m=True):
    if blocked_m:
        return pl.BlockSpec(shape, lambda i: (i,) + (0,) * (len(shape) - 1))
    return pl.BlockSpec(shape, lambda i: (0,) * len(shape))


def _make_tc_first():
    def body(x_ref, w_ref, da_ref, db_ref, o0_ref, o1_ref):
        s = lax.rsqrt(da_ref[...] + db_ref[...])
        g = jnp.dot(x_ref[...], w_ref[...],
                    preferred_element_type=jnp.float32) * s
        o0_ref[...] = g[:, :128]
        o1_ref[...] = g[:, 128:]

    return pl.pallas_call(
        body,
        grid=(N_NODES // BM,),
        in_specs=[
            _bspec((BM, 256)),
            _bspec((256, 256), blocked_m=False),
            _bspec((BM, 1)),
            _bspec((BM, 1)),
        ],
        out_specs=[_bspec((BM, 128)), _bspec((BM, 128))],
        out_shape=[jax.ShapeDtypeStruct((N_NODES, 128), jnp.float32)] * 2,
    )


def _make_tc_step256(dout, relu):
    h_out = dout // 128

    def body(p_ref, da_ref, db_ref, b_ref, w_ref, *o_refs):
        s = lax.rsqrt(da_ref[...] + db_ref[...])
        p = p_ref[...]
        t = s * jnp.concatenate([p[0], p[1]], axis=1) + b_ref[...]
        if relu:
            t = jnp.maximum(t, 0.0)
        g = jnp.dot(t, w_ref[...], preferred_element_type=jnp.float32) * s
        for h in range(h_out):
            o_refs[h][...] = g[:, h * 128:(h + 1) * 128]

    return pl.pallas_call(
        body,
        grid=(N_NODES // BM,),
        in_specs=[
            pl.BlockSpec((2, BM, 128), lambda i: (0, i, 0)),
            _bspec((BM, 1)),
            _bspec((BM, 1)),
            _bspec((1, 256), blocked_m=False),
            _bspec((256, dout), blocked_m=False),
        ],
        out_specs=[_bspec((BM, 128))] * h_out,
        out_shape=[jax.ShapeDtypeStruct((N_NODES, 128), jnp.float32)] * h_out,
    )


def _make_tc_step128(relu):
    def body(p_ref, da_ref, db_ref, b_ref, w_ref, o0_ref, o1_ref):
        s = lax.rsqrt(da_ref[...] + db_ref[...])
        p = p_ref[...]
        t = s * (p[0] + p[1]) + b_ref[...]
        if relu:
            t = jnp.maximum(t, 0.0)
        g = jnp.dot(t, w_ref[...], preferred_element_type=jnp.float32) * s
        o0_ref[...] = g[:, :128]
        o1_ref[...] = g[:, 128:]

    return pl.pallas_call(
        body,
        grid=(N_NODES // BM,),
        in_specs=[
            pl.BlockSpec((2, BM, 128), lambda i: (0, i, 0)),
            _bspec((BM, 1)),
            _bspec((BM, 1)),
            _bspec((1, 128), blocked_m=False),
            _bspec((128, 256), blocked_m=False),
        ],
        out_specs=[_bspec((BM, 128)), _bspec((BM, 128))],
        out_shape=[jax.ShapeDtypeStruct((N_NODES, 128), jnp.float32)] * 2,
    )


def _make_tc_final():
    def body(p_ref, da_ref, db_ref, b_ref, o_ref):
        s = lax.rsqrt(da_ref[...] + db_ref[...])
        p = p_ref[...]
        o_ref[...] = s * jnp.concatenate([p[0], p[1]], axis=1) + b_ref[...]

    return pl.pallas_call(
        body,
        grid=(N_NODES // BM,),
        in_specs=[
            pl.BlockSpec((2, BM, 128), lambda i: (0, i, 0)),
            _bspec((BM, 1)),
            _bspec((BM, 1)),
            _bspec((1, 256), blocked_m=False),
        ],
        out_specs=_bspec((BM, 256)),
        out_shape=jax.ShapeDtypeStruct((N_NODES, 256), jnp.float32),
    )


def kernel(x, edge_index, W1, b1, W2, b2, W3, b3, W4, b4, W5, b5, W6, b6):
    src = edge_index[0].astype(jnp.int32)
    dst = edge_index[1].astype(jnp.int32)
    e_pad = E_PAD - src.shape[0]
    src_p = jnp.concatenate(
        [src, (jnp.arange(e_pad, dtype=jnp.int32) * 64) % N_NODES])
    dst_p = jnp.concatenate(
        [dst, N_NODES + (jnp.arange(e_pad, dtype=jnp.int32) % (N_PAD - N_NODES))])

    ones128 = jnp.ones((N_NODES, 128), jnp.float32)
    zeros128 = jnp.zeros((N_NODES, 128), jnp.float32)

    sc_full = _make_scatter_full()
    sc_part = _make_scatter_partial()
    tc_first = _make_tc_first()
    tc_256_256 = _make_tc_step256(256, relu=True)
    tc_256_128 = _make_tc_step256(128, relu=True)
    tc_128_256 = _make_tc_step128(relu=False)
    tc_final = _make_tc_final()

    deg = sc_part(ones128, src_p, dst_p, zeros128)
    da = deg[0, :, 0:1]
    db = deg[1, :, 0:1]

    g0, g1 = tc_first(x, W1, da, db)
    p = sc_full(g0, g1, src_p, dst_p)
    g0, g1 = tc_256_256(p, da, db, b1.reshape(1, -1), W2)
    p = sc_full(g0, g1, src_p, dst_p)
    (g3,) = tc_256_128(p, da, db, b2.reshape(1, -1), W3)
    q = sc_part(g3, src_p, dst_p, zeros128)
    g0, g1 = tc_128_256(q, da, db, b3.reshape(1, -1), W4)
    p = sc_full(g0, g1, src_p, dst_p)
    g0, g1 = tc_256_256(p, da, db, b4.reshape(1, -1), W5)
    p = sc_full(g0, g1, src_p, dst_p)
    g0, g1 = tc_256_256(p, da, db, b5.reshape(1, -1), W6)
    p = sc_full(g0, g1, src_p, dst_p)
    out = tc_final(p, da, db, b6.reshape(1, -1))
    return out
